# baseline (speedup 1.0000x reference)
import functools

import jax
import jax.numpy as jnp
from jax import lax
from jax.experimental import pallas as pl
from jax.experimental.pallas import tpu as pltpu
from jax.experimental.pallas import tpu_sc as plsc

_STUFF_THRESH = 10
_INST_BASE = 11
_L = 16
_NC = 2
_NS = 16
_NW = _NC * _NS


@functools.cache
def _make_sc_call(num_gt, hw, chunk):
    P = hw // _NW
    nchunk = P // chunk
    assert P % chunk == 0 and nchunk % 2 == 0

    mesh = plsc.VectorSubcoreMesh(
        core_axis_name="c", subcore_axis_name="s",
        num_cores=_NC, num_subcores=_NS,
    )

    def body(segs_hbm, masks_hbm, out_hbm, masks_v, segs_v, out_v,
             si0, si1, so0, so1):
        wid = lax.axis_index("s") * _NC + lax.axis_index("c")
        base0 = wid * P
        in_sems = (si0, si1)
        out_sems = (so0, so1)

        def in_copies(k, b):
            start = base0 + k * chunk
            cps = [
                pltpu.make_async_copy(
                    masks_hbm.at[pl.ds(i * hw + start, chunk)],
                    masks_v.at[b, i],
                    in_sems[b],
                )
                for i in range(num_gt)
            ]
            cps.append(
                pltpu.make_async_copy(
                    segs_hbm.at[pl.ds(start, chunk)], segs_v.at[b], in_sems[b]
                )
            )
            return cps

        for cp in in_copies(0, 0):
            cp.start()
        for cp in in_copies(1, 1):
            cp.start()

        def pair(p, carry):
            k0 = p * 2
            for b in range(2):
                k = k0 + b
                for cp in in_copies(k, b):
                    cp.wait()

                @pl.when(k >= 2)
                def _():
                    pltpu.make_async_copy(
                        out_v.at[b], out_hbm.at[pl.ds(base0, chunk)], out_sems[b]
                    ).wait()

                def inner(v, c):
                    off = v * _L
                    acc = jnp.full((_L,), -1, jnp.int32)
                    for i in range(num_gt):
                        m = masks_v[b, i, pl.ds(off, _L)]
                        acc = jnp.where(m != 0.0, i, acc)
                    seg = segs_v[b, pl.ds(off, _L)]
                    stuff = jnp.where(
                        (seg <= _STUFF_THRESH) | (seg >= 255), seg, 255
                    )
                    out_v[b, pl.ds(off, _L)] = jnp.where(
                        acc >= 0, acc + _INST_BASE, stuff
                    )
                    return c

                lax.fori_loop(0, chunk // _L, inner, 0)

                pltpu.make_async_copy(
                    out_v.at[b],
                    out_hbm.at[pl.ds(base0 + k * chunk, chunk)],
                    out_sems[b],
                ).start()

                @pl.when(k + 2 < nchunk)
                def _():
                    for cp in in_copies(k + 2, b):
                        cp.start()
            return carry

        lax.fori_loop(0, nchunk // 2, pair, 0)

        for b in range(2):
            pltpu.make_async_copy(
                out_v.at[b], out_hbm.at[pl.ds(base0, chunk)], out_sems[b]
            ).wait()

    return pl.kernel(
        body,
        out_type=jax.ShapeDtypeStruct((hw,), jnp.int32),
        mesh=mesh,
        scratch_types=[
            pltpu.VMEM((2, num_gt, chunk), jnp.float32),
            pltpu.VMEM((2, chunk), jnp.int32),
            pltpu.VMEM((2, chunk), jnp.int32),
            pltpu.SemaphoreType.DMA,
            pltpu.SemaphoreType.DMA,
            pltpu.SemaphoreType.DMA,
            pltpu.SemaphoreType.DMA,
        ],
    )


def kernel(gt_segs, gt_masks):
    _, H, W = gt_segs.shape
    num_gt = gt_masks.shape[0]
    hw = H * W
    segs_flat = gt_segs.reshape(hw)
    masks_flat = gt_masks.reshape(num_gt * hw)
    out = _make_sc_call(num_gt, hw, 1024)(segs_flat, masks_flat)
    return out.reshape(1, H, W)


# speedup vs baseline: 2.7320x; 2.7320x over previous
import functools

import jax
import jax.numpy as jnp
from jax import lax
from jax.experimental import pallas as pl
from jax.experimental.pallas import tpu as pltpu
from jax.experimental.pallas import tpu_sc as plsc

_STUFF_THRESH = 10
_INST_BASE = 11
_L = 16
_NC = 2
_NS = 16
_NW = _NC * _NS
_TR = 8
_TC = 128


@functools.cache
def _make_sc_call(num_gt, H, W):
    col_tiles = W // _TC
    total_tiles = (H // _TR) * col_tiles
    nchunk = total_tiles // _NW
    assert total_tiles % _NW == 0 and nchunk % 2 == 0

    mesh = plsc.VectorSubcoreMesh(
        core_axis_name="c", subcore_axis_name="s",
        num_cores=_NC, num_subcores=_NS,
    )

    def body(segs_hbm, masks_hbm, out_hbm, masks_v, segs_v, out_v,
             si0, si1, so0, so1):
        wid = lax.axis_index("s") * _NC + lax.axis_index("c")
        t0 = wid * nchunk
        in_sems = (si0, si1)
        out_sems = (so0, so1)

        def tile_origin(k):
            t = t0 + k
            rb = t // col_tiles
            ct = t % col_tiles
            return rb * _TR, ct * _TC

        def in_copies(k, b):
            r0, c0 = tile_origin(k)
            cps = [
                pltpu.make_async_copy(
                    masks_hbm.at[i, pl.ds(r0, _TR), pl.ds(c0, _TC)],
                    masks_v.at[b, i],
                    in_sems[b],
                )
                for i in range(num_gt)
            ]
            cps.append(
                pltpu.make_async_copy(
                    segs_hbm.at[0, pl.ds(r0, _TR), pl.ds(c0, _TC)],
                    segs_v.at[b],
                    in_sems[b],
                )
            )
            return cps

        def out_copy(k, b):
            r0, c0 = tile_origin(k)
            return pltpu.make_async_copy(
                out_v.at[b],
                out_hbm.at[0, pl.ds(r0, _TR), pl.ds(c0, _TC)],
                out_sems[b],
            )

        for cp in in_copies(0, 0):
            cp.start()
        for cp in in_copies(1, 1):
            cp.start()

        def pair(p, carry):
            k0 = p * 2
            for b in range(2):
                k = k0 + b
                for cp in in_copies(k, b):
                    cp.wait()

                @pl.when(k >= 2)
                def _():
                    out_copy(k, b).wait()

                def inner(v, c):
                    r = v // (_TC // _L)
                    off = (v % (_TC // _L)) * _L
                    acc = jnp.full((_L,), -1, jnp.int32)
                    for i in range(num_gt):
                        m = masks_v[b, i, r, pl.ds(off, _L)]
                        acc = jnp.where(m != 0.0, i, acc)
                    seg = segs_v[b, r, pl.ds(off, _L)]
                    stuff = jnp.where(
                        (seg <= _STUFF_THRESH) | (seg >= 255), seg, 255
                    )
                    out_v[b, r, pl.ds(off, _L)] = jnp.where(
                        acc >= 0, acc + _INST_BASE, stuff
                    )
                    return c

                lax.fori_loop(0, _TR * (_TC // _L), inner, 0)

                out_copy(k, b).start()

                @pl.when(k + 2 < nchunk)
                def _():
                    for cp in in_copies(k + 2, b):
                        cp.start()
            return carry

        lax.fori_loop(0, nchunk // 2, pair, 0)

        for b in range(2):
            out_copy(0, b).wait()

    return pl.kernel(
        body,
        out_type=jax.ShapeDtypeStruct((1, H, W), jnp.int32),
        mesh=mesh,
        scratch_types=[
            pltpu.VMEM((2, num_gt, _TR, _TC), jnp.float32),
            pltpu.VMEM((2, _TR, _TC), jnp.int32),
            pltpu.VMEM((2, _TR, _TC), jnp.int32),
            pltpu.SemaphoreType.DMA,
            pltpu.SemaphoreType.DMA,
            pltpu.SemaphoreType.DMA,
            pltpu.SemaphoreType.DMA,
        ],
        compiler_params=pltpu.CompilerParams(use_tc_tiling_on_sc=True),
    )


def kernel(gt_segs, gt_masks):
    _, H, W = gt_segs.shape
    num_gt = gt_masks.shape[0]
    return _make_sc_call(num_gt, H, W)(gt_segs, gt_masks)


# speedup vs baseline: 3.7674x; 1.3790x over previous
import functools

import jax
import jax.numpy as jnp
from jax import lax
from jax.experimental import pallas as pl
from jax.experimental.pallas import tpu as pltpu
from jax.experimental.pallas import tpu_sc as plsc

_STUFF_THRESH = 10
_INST_BASE = 11
_L = 16
_NC = 2
_NS = 16
_NW = _NC * _NS
_TR = 8
_TC = 128

_SC_ROWS = 448


@functools.cache
def _make_sc_call(num_gt, H, W, row0, rows):
    col_tiles = W // _TC
    total_tiles = (rows // _TR) * col_tiles
    nchunk = total_tiles // _NW
    assert total_tiles % _NW == 0 and nchunk % 2 == 0
    rb0 = row0 // _TR

    mesh = plsc.VectorSubcoreMesh(
        core_axis_name="c", subcore_axis_name="s",
        num_cores=_NC, num_subcores=_NS,
    )

    def body(segs_hbm, masks_hbm, out_hbm, masks_v, segs_v, out_v,
             si0, si1, so0, so1):
        wid = lax.axis_index("s") * _NC + lax.axis_index("c")
        t0 = wid * nchunk
        in_sems = (si0, si1)
        out_sems = (so0, so1)

        def tile_origin(k):
            t = t0 + k
            rb = t // col_tiles
            ct = t % col_tiles
            return rb * _TR, ct * _TC

        def in_copies(k, b):
            r0, c0 = tile_origin(k)
            cps = [
                pltpu.make_async_copy(
                    masks_hbm.at[i, pl.ds(row0 + r0, _TR), pl.ds(c0, _TC)],
                    masks_v.at[b, i],
                    in_sems[b],
                )
                for i in range(num_gt)
            ]
            cps.append(
                pltpu.make_async_copy(
                    segs_hbm.at[0, pl.ds(row0 + r0, _TR), pl.ds(c0, _TC)],
                    segs_v.at[b],
                    in_sems[b],
                )
            )
            return cps

        def out_copy(k, b):
            r0, c0 = tile_origin(k)
            return pltpu.make_async_copy(
                out_v.at[b],
                out_hbm.at[0, pl.ds(r0, _TR), pl.ds(c0, _TC)],
                out_sems[b],
            )

        for cp in in_copies(0, 0):
            cp.start()
        for cp in in_copies(1, 1):
            cp.start()

        def pair(p, carry):
            k0 = p * 2
            for b in range(2):
                k = k0 + b
                for cp in in_copies(k, b):
                    cp.wait()

                @pl.when(k >= 2)
                def _():
                    out_copy(k, b).wait()

                def inner(v, c):
                    r = v // (_TC // _L)
                    off = (v % (_TC // _L)) * _L
                    acc = jnp.full((_L,), -1, jnp.int32)
                    for i in range(num_gt):
                        m = masks_v[b, i, r, pl.ds(off, _L)]
                        acc = jnp.where(m != 0.0, i, acc)
                    seg = segs_v[b, r, pl.ds(off, _L)]
                    stuff = jnp.where(
                        (seg <= _STUFF_THRESH) | (seg >= 255), seg, 255
                    )
                    out_v[b, r, pl.ds(off, _L)] = jnp.where(
                        acc >= 0, acc + _INST_BASE, stuff
                    )
                    return c

                lax.fori_loop(0, _TR * (_TC // _L), inner, 0)

                out_copy(k, b).start()

                @pl.when(k + 2 < nchunk)
                def _():
                    for cp in in_copies(k + 2, b):
                        cp.start()
            return carry

        lax.fori_loop(0, nchunk // 2, pair, 0)

        for b in range(2):
            out_copy(0, b).wait()

    return pl.kernel(
        body,
        out_type=jax.ShapeDtypeStruct((1, rows, W), jnp.int32),
        mesh=mesh,
        scratch_types=[
            pltpu.VMEM((2, num_gt, _TR, _TC), jnp.float32),
            pltpu.VMEM((2, _TR, _TC), jnp.int32),
            pltpu.VMEM((2, _TR, _TC), jnp.int32),
            pltpu.SemaphoreType.DMA,
            pltpu.SemaphoreType.DMA,
            pltpu.SemaphoreType.DMA,
            pltpu.SemaphoreType.DMA,
        ],
        compiler_params=pltpu.CompilerParams(use_tc_tiling_on_sc=True),
    )


def _tc_body(segs_ref, masks_ref, out_ref):
    num_gt = masks_ref.shape[0]
    seg = segs_ref[0]
    best = jnp.full(seg.shape, -1, jnp.int32)
    for i in range(num_gt):
        best = jnp.where(masks_ref[i] != 0.0, i, best)
    stuff = jnp.where((seg <= _STUFF_THRESH) | (seg >= 255), seg, 255)
    out_ref[0] = jnp.where(best >= 0, best + _INST_BASE, stuff)


@functools.cache
def _make_tc_call(num_gt, H, W, rows, rb):
    grid = (rows // rb,)
    return pl.pallas_call(
        _tc_body,
        grid=grid,
        in_specs=[
            pl.BlockSpec((1, rb, W), lambda i: (0, i, 0)),
            pl.BlockSpec((num_gt, rb, W), lambda i: (0, i, 0)),
        ],
        out_specs=pl.BlockSpec((1, rb, W), lambda i: (0, i, 0)),
        out_shape=jax.ShapeDtypeStruct((1, rows, W), jnp.int32),
        compiler_params=pltpu.CompilerParams(
            dimension_semantics=("arbitrary",),
        ),
    )


def kernel(gt_segs, gt_masks):
    _, H, W = gt_segs.shape
    num_gt = gt_masks.shape[0]
    sc_rows = _SC_ROWS
    tc_rows = H - sc_rows
    out_sc = _make_sc_call(num_gt, H, W, tc_rows, sc_rows)(gt_segs, gt_masks)
    out_tc = _make_tc_call(num_gt, H, W, tc_rows, 8)(gt_segs, gt_masks)
    return jnp.concatenate([out_tc, out_sc], axis=1)


# speedup vs baseline: 3.8075x; 1.0106x over previous
import functools

import jax
import jax.numpy as jnp
from jax import lax
from jax.experimental import pallas as pl
from jax.experimental.pallas import tpu as pltpu
from jax.experimental.pallas import tpu_sc as plsc

_STUFF_THRESH = 10
_INST_BASE = 11
_L = 16
_NC = 2
_NS = 16
_NW = _NC * _NS
_TR = 8
_TC = 128

_SC_ROWS = 480


@functools.cache
def _make_sc_call(num_gt, H, W, row0, rows):
    col_tiles = W // _TC
    total_tiles = (rows // _TR) * col_tiles
    nchunk = total_tiles // _NW
    assert total_tiles % _NW == 0 and nchunk % 2 == 0
    rb0 = row0 // _TR

    mesh = plsc.VectorSubcoreMesh(
        core_axis_name="c", subcore_axis_name="s",
        num_cores=_NC, num_subcores=_NS,
    )

    def body(segs_hbm, masks_hbm, out_hbm, masks_v, segs_v, out_v,
             si0, si1, so0, so1):
        wid = lax.axis_index("s") * _NC + lax.axis_index("c")
        t0 = wid * nchunk
        in_sems = (si0, si1)
        out_sems = (so0, so1)

        def tile_origin(k):
            t = t0 + k
            rb = t // col_tiles
            ct = t % col_tiles
            return rb * _TR, ct * _TC

        def in_copies(k, b):
            r0, c0 = tile_origin(k)
            cps = [
                pltpu.make_async_copy(
                    masks_hbm.at[i, pl.ds(row0 + r0, _TR), pl.ds(c0, _TC)],
                    masks_v.at[b, i],
                    in_sems[b],
                )
                for i in range(num_gt)
            ]
            cps.append(
                pltpu.make_async_copy(
                    segs_hbm.at[0, pl.ds(row0 + r0, _TR), pl.ds(c0, _TC)],
                    segs_v.at[b],
                    in_sems[b],
                )
            )
            return cps

        def out_copy(k, b):
            r0, c0 = tile_origin(k)
            return pltpu.make_async_copy(
                out_v.at[b],
                out_hbm.at[0, pl.ds(r0, _TR), pl.ds(c0, _TC)],
                out_sems[b],
            )

        for cp in in_copies(0, 0):
            cp.start()
        for cp in in_copies(1, 1):
            cp.start()

        def pair(p, carry):
            k0 = p * 2
            for b in range(2):
                k = k0 + b
                for cp in in_copies(k, b):
                    cp.wait()

                @pl.when(k >= 2)
                def _():
                    out_copy(k, b).wait()

                def inner(v, c):
                    r = v // (_TC // _L)
                    off = (v % (_TC // _L)) * _L
                    acc = jnp.full((_L,), -1, jnp.int32)
                    for i in range(num_gt):
                        m = masks_v[b, i, r, pl.ds(off, _L)]
                        acc = jnp.where(m != 0.0, i, acc)
                    seg = segs_v[b, r, pl.ds(off, _L)]
                    stuff = jnp.where(
                        (seg <= _STUFF_THRESH) | (seg >= 255), seg, 255
                    )
                    out_v[b, r, pl.ds(off, _L)] = jnp.where(
                        acc >= 0, acc + _INST_BASE, stuff
                    )
                    return c

                lax.fori_loop(0, _TR * (_TC // _L), inner, 0)

                out_copy(k, b).start()

                @pl.when(k + 2 < nchunk)
                def _():
                    for cp in in_copies(k + 2, b):
                        cp.start()
            return carry

        lax.fori_loop(0, nchunk // 2, pair, 0)

        for b in range(2):
            out_copy(0, b).wait()

    return pl.kernel(
        body,
        out_type=jax.ShapeDtypeStruct((1, rows, W), jnp.int32),
        mesh=mesh,
        scratch_types=[
            pltpu.VMEM((2, num_gt, _TR, _TC), jnp.float32),
            pltpu.VMEM((2, _TR, _TC), jnp.int32),
            pltpu.VMEM((2, _TR, _TC), jnp.int32),
            pltpu.SemaphoreType.DMA,
            pltpu.SemaphoreType.DMA,
            pltpu.SemaphoreType.DMA,
            pltpu.SemaphoreType.DMA,
        ],
        compiler_params=pltpu.CompilerParams(use_tc_tiling_on_sc=True),
    )


def _tc_body(segs_ref, masks_ref, out_ref):
    num_gt = masks_ref.shape[0]
    seg = segs_ref[0]
    best = jnp.full(seg.shape, -1, jnp.int32)
    for i in range(num_gt):
        best = jnp.where(masks_ref[i] != 0.0, i, best)
    stuff = jnp.where((seg <= _STUFF_THRESH) | (seg >= 255), seg, 255)
    out_ref[0] = jnp.where(best >= 0, best + _INST_BASE, stuff)


@functools.cache
def _make_tc_call(num_gt, H, W, rows, rb):
    grid = (rows // rb,)
    return pl.pallas_call(
        _tc_body,
        grid=grid,
        in_specs=[
            pl.BlockSpec((1, rb, W), lambda i: (0, i, 0)),
            pl.BlockSpec((num_gt, rb, W), lambda i: (0, i, 0)),
        ],
        out_specs=pl.BlockSpec((1, rb, W), lambda i: (0, i, 0)),
        out_shape=jax.ShapeDtypeStruct((1, rows, W), jnp.int32),
        compiler_params=pltpu.CompilerParams(
            dimension_semantics=("arbitrary",),
        ),
    )


def kernel(gt_segs, gt_masks):
    _, H, W = gt_segs.shape
    num_gt = gt_masks.shape[0]
    sc_rows = _SC_ROWS
    tc_rows = H - sc_rows
    out_sc = _make_sc_call(num_gt, H, W, tc_rows, sc_rows)(gt_segs, gt_masks)
    out_tc = _make_tc_call(num_gt, H, W, tc_rows, 8)(gt_segs, gt_masks)
    return jnp.concatenate([out_tc, out_sc], axis=1)
